# baseline (speedup 1.0000x reference)
import functools

import jax
import jax.numpy as jnp
from jax import lax
from jax.experimental import pallas as pl
from jax.experimental.pallas import tpu as pltpu
from jax.experimental.pallas import tpu_sc as plsc

N_ROWS = 1_000_000
N_COLS = 64
BLK = 20000
NB = N_ROWS // BLK

SC_CORES = 2
SC_SUBCORES = 16
NW = SC_CORES * SC_SUBCORES
UNITS = N_ROWS // 16
Q, R = divmod(UNITS, NW)
MAXV = (Q + 1) * 16


NSTREAM = 4
SUB = BLK // NSTREAM


def _dense_body(*refs):
    lrefs = refs[:NSTREAM]
    labref = refs[NSTREAM]
    oref = refs[NSTREAM + 1]
    for k in range(NSTREAM):
        xt = lrefs[k][...].T
        lbl = labref[0, 0, k * SUB:(k + 1) * SUB].astype(jnp.float32)
        m = jnp.max(xt, axis=0)
        ii = lax.broadcasted_iota(jnp.int32, xt.shape, 0).astype(jnp.float32)
        pred = jnp.min(
            jnp.where(xt == m[None, :], ii, jnp.float32(N_COLS)), axis=0
        )
        s = jnp.sum(jnp.exp(xt - m[None, :]), axis=0)
        conf = 1.0 / s
        oref[0, 0, k * SUB:(k + 1) * SUB] = jnp.where(pred == lbl, conf, -conf)


def _dense_call(logits, lab3):
    lspecs = [
        pl.BlockSpec((SUB, N_COLS), lambda i, k=k: (NSTREAM * i + k, 0))
        for k in range(NSTREAM)
    ]
    return pl.pallas_call(
        _dense_body,
        grid=(NB,),
        in_specs=lspecs + [pl.BlockSpec((1, 1, BLK), lambda i: (i, 0, 0))],
        out_specs=pl.BlockSpec((1, 1, BLK), lambda i: (i, 0, 0)),
        out_shape=jax.ShapeDtypeStruct((NB, 1, BLK), jnp.float32),
    )(*([logits] * NSTREAM), lab3)


NSET = 3
SEG = NSET * 240


def _sc_bin_body(v_hbm, bnd_hbm, out_hbm, buf, bndv, cnt, sumc, suma):
    c = lax.axis_index("c")
    s_ = lax.axis_index("s")
    w = s_ * SC_CORES + c
    start = (w * Q + jnp.minimum(w, R)) * 16

    pltpu.sync_copy(bnd_hbm, bndv)
    zero16 = jnp.zeros((16,), jnp.float32)
    for k in range(SEG // 16):
        cnt[pl.ds(k * 16, 16)] = zero16
        sumc[pl.ds(k * 16, 16)] = zero16
        suma[pl.ds(k * 16, 16)] = zero16

    pltpu.sync_copy(v_hbm.at[pl.ds(start, Q * 16)], buf.at[pl.ds(0, Q * 16)])

    @pl.when(w < R)
    def _():
        pltpu.sync_copy(
            v_hbm.at[pl.ds(start + Q * 16, 16)], buf.at[pl.ds(Q * 16, 16)]
        )

    iota16 = lax.iota(jnp.int32, 16)
    ones16 = jnp.ones((16,), jnp.float32)

    def unit(off, setk):
        v = buf[pl.ds(off, 16)]
        confv = jnp.abs(v)
        accf = jnp.where(v > 0, ones16, zero16)
        g = jnp.minimum((confv * 15.0).astype(jnp.int32), 14)
        lo = plsc.load_gather(bndv, [g])
        g = jnp.where(confv <= lo, g - 1, g)
        hi = plsc.load_gather(bndv, [g + 1])
        g = jnp.where(confv > hi, g + 1, g)
        slot = g * 16 + iota16 + setk * 240
        plsc.addupdate_scatter(cnt, [slot], ones16)
        plsc.addupdate_scatter(sumc, [slot], confv)
        plsc.addupdate_scatter(suma, [slot], accf)

    def body(i, carry):
        base = i * (9 * 16)
        for k in range(9):
            unit(base + k * 16, k % NSET)
        return carry

    lax.fori_loop(0, Q // 9, body, 0)

    @pl.when(w < R)
    def _():
        unit(Q * 16, 0)

    pltpu.sync_copy(cnt, out_hbm.at[pl.ds(w * (3 * SEG), SEG)])
    pltpu.sync_copy(sumc, out_hbm.at[pl.ds(w * (3 * SEG) + SEG, SEG)])
    pltpu.sync_copy(suma, out_hbm.at[pl.ds(w * (3 * SEG) + 2 * SEG, SEG)])


@functools.lru_cache(maxsize=1)
def _make_sc_bin():
    mesh = plsc.VectorSubcoreMesh(
        core_axis_name="c", subcore_axis_name="s", num_cores=SC_CORES
    )
    return pl.kernel(
        _sc_bin_body,
        mesh=mesh,
        compiler_params=pltpu.CompilerParams(needs_layout_passes=False),
        out_type=jax.ShapeDtypeStruct((NW * 3 * SEG,), jnp.float32),
        scratch_types=[
            pltpu.VMEM((MAXV,), jnp.float32),
            pltpu.VMEM((16,), jnp.float32),
            pltpu.VMEM((SEG,), jnp.float32),
            pltpu.VMEM((SEG,), jnp.float32),
            pltpu.VMEM((SEG,), jnp.float32),
        ],
    )


def _final_body(pref, oref):
    p = pref[...]
    s = jnp.sum(p, axis=0)
    cnt240 = s[0:1] + s[1:2] + s[2:3]
    sumc240 = s[3:4] + s[4:5] + s[5:6]
    suma240 = s[6:7] + s[7:8] + s[8:9]
    grp = lax.broadcasted_iota(jnp.int32, (15, 240), 1) // 16
    row = lax.broadcasted_iota(jnp.int32, (15, 240), 0)
    onehot = (grp == row).astype(jnp.float32)
    count = jnp.sum(onehot * cnt240, axis=1)
    sumc = jnp.sum(onehot * sumc240, axis=1)
    suma = jnp.sum(onehot * suma240, axis=1)
    denom = jnp.maximum(count, 1.0)
    contrib = jnp.where(
        count > 0,
        jnp.abs(sumc / denom - suma / denom) * (count / float(N_ROWS)),
        0.0,
    )
    oref[...] = jnp.sum(contrib).reshape(1, 1)


def _final_call(partials):
    return pl.pallas_call(
        _final_body,
        out_shape=jax.ShapeDtypeStruct((1, 1), jnp.float32),
    )(partials)


def kernel(logits, labels):
    lab3 = labels.astype(jnp.int32).reshape(NB, 1, BLK)
    venc = _dense_call(logits, lab3)
    vflat = venc.reshape(N_ROWS)
    bnd = jnp.linspace(0.0, 1.0, 16, dtype=jnp.float32)
    partials = _make_sc_bin()(vflat, bnd)
    ece = _final_call(partials.reshape(NW, 9, 240))
    return ece.reshape(1)


# speedup vs baseline: 1.0709x; 1.0709x over previous
import functools

import jax
import jax.numpy as jnp
from jax import lax
from jax.experimental import pallas as pl
from jax.experimental.pallas import tpu as pltpu
from jax.experimental.pallas import tpu_sc as plsc

N_ROWS = 1_000_000
N_COLS = 64
BLK = 20000
NB = N_ROWS // BLK

SC_CORES = 2
SC_SUBCORES = 16
NW = SC_CORES * SC_SUBCORES
UNITS = N_ROWS // 16
Q, R = divmod(UNITS, NW)
MAXV = (Q + 1) * 16


NSTREAM = 4
SUB = BLK // NSTREAM


def _dense_body(*refs):
    lrefs = refs[:NSTREAM]
    labref = refs[NSTREAM]
    oref = refs[NSTREAM + 1]
    for k in range(NSTREAM):
        xt = lrefs[k][...].T
        lbl = labref[0, 0, k * SUB:(k + 1) * SUB].astype(jnp.float32)
        m = jnp.max(xt, axis=0)
        ii = lax.broadcasted_iota(jnp.int32, xt.shape, 0).astype(jnp.float32)
        pred = jnp.min(
            jnp.where(xt == m[None, :], ii, jnp.float32(N_COLS)), axis=0
        )
        s = jnp.sum(jnp.exp(xt - m[None, :]), axis=0)
        conf = 1.0 / s
        oref[0, 0, k * SUB:(k + 1) * SUB] = jnp.where(pred == lbl, conf, -conf)


def _dense_call(logits, lab3):
    lspecs = [
        pl.BlockSpec((SUB, N_COLS), lambda i, k=k: (NSTREAM * i + k, 0))
        for k in range(NSTREAM)
    ]
    return pl.pallas_call(
        _dense_body,
        grid=(NB,),
        in_specs=lspecs + [pl.BlockSpec((1, 1, BLK), lambda i: (i, 0, 0))],
        out_specs=pl.BlockSpec((1, 1, BLK), lambda i: (i, 0, 0)),
        out_shape=jax.ShapeDtypeStruct((NB, 1, BLK), jnp.float32),
    )(*([logits] * NSTREAM), lab3)


NSET = 3
SEG = NSET * 240


def _sc_bin_body(v_hbm, bnd_hbm, out_hbm, buf, bndv, cnt, sumc, suma):
    c = lax.axis_index("c")
    s_ = lax.axis_index("s")
    w = s_ * SC_CORES + c
    start = (w * Q + jnp.minimum(w, R)) * 16

    pltpu.sync_copy(bnd_hbm, bndv)
    zero16 = jnp.zeros((16,), jnp.float32)
    for k in range(SEG // 16):
        cnt[pl.ds(k * 16, 16)] = zero16
        sumc[pl.ds(k * 16, 16)] = zero16
        suma[pl.ds(k * 16, 16)] = zero16

    pltpu.sync_copy(v_hbm.at[pl.ds(start, Q * 16)], buf.at[pl.ds(0, Q * 16)])

    @pl.when(w < R)
    def _():
        pltpu.sync_copy(
            v_hbm.at[pl.ds(start + Q * 16, 16)], buf.at[pl.ds(Q * 16, 16)]
        )

    iota16 = lax.iota(jnp.int32, 16)
    ones16 = jnp.ones((16,), jnp.float32)

    def unit(off, setk):
        v = buf[pl.ds(off, 16)]
        confv = jnp.abs(v)
        accf = jnp.where(v > 0, ones16, zero16)
        g = jnp.minimum((confv * 15.0).astype(jnp.int32), 14)
        lo = plsc.load_gather(bndv, [g])
        g = jnp.where(confv <= lo, g - 1, g)
        hi = plsc.load_gather(bndv, [g + 1])
        g = jnp.where(confv > hi, g + 1, g)
        slot = g * 16 + iota16 + setk * jnp.int32(240)
        plsc.addupdate_scatter(cnt, [slot], ones16)
        plsc.addupdate_scatter(sumc, [slot], confv)
        plsc.addupdate_scatter(suma, [slot], accf)

    @plsc.parallel_loop(0, Q, step=1, unroll=9)
    def _(i):
        unit(i * 16, i % NSET)

    @pl.when(w < R)
    def _():
        unit(Q * 16, 0)

    pltpu.sync_copy(cnt, out_hbm.at[pl.ds(w * (3 * SEG), SEG)])
    pltpu.sync_copy(sumc, out_hbm.at[pl.ds(w * (3 * SEG) + SEG, SEG)])
    pltpu.sync_copy(suma, out_hbm.at[pl.ds(w * (3 * SEG) + 2 * SEG, SEG)])


@functools.lru_cache(maxsize=1)
def _make_sc_bin():
    mesh = plsc.VectorSubcoreMesh(
        core_axis_name="c", subcore_axis_name="s", num_cores=SC_CORES
    )
    return pl.kernel(
        _sc_bin_body,
        mesh=mesh,
        compiler_params=pltpu.CompilerParams(needs_layout_passes=False),
        out_type=jax.ShapeDtypeStruct((NW * 3 * SEG,), jnp.float32),
        scratch_types=[
            pltpu.VMEM((MAXV,), jnp.float32),
            pltpu.VMEM((16,), jnp.float32),
            pltpu.VMEM((SEG,), jnp.float32),
            pltpu.VMEM((SEG,), jnp.float32),
            pltpu.VMEM((SEG,), jnp.float32),
        ],
    )


def _final_body(pref, oref):
    p = pref[...]
    s = jnp.sum(p, axis=0)
    cnt240 = s[0:1] + s[1:2] + s[2:3]
    sumc240 = s[3:4] + s[4:5] + s[5:6]
    suma240 = s[6:7] + s[7:8] + s[8:9]
    grp = lax.broadcasted_iota(jnp.int32, (15, 240), 1) // 16
    row = lax.broadcasted_iota(jnp.int32, (15, 240), 0)
    onehot = (grp == row).astype(jnp.float32)
    count = jnp.sum(onehot * cnt240, axis=1)
    sumc = jnp.sum(onehot * sumc240, axis=1)
    suma = jnp.sum(onehot * suma240, axis=1)
    denom = jnp.maximum(count, 1.0)
    contrib = jnp.where(
        count > 0,
        jnp.abs(sumc / denom - suma / denom) * (count / float(N_ROWS)),
        0.0,
    )
    oref[...] = jnp.sum(contrib).reshape(1, 1)


def _final_call(partials):
    return pl.pallas_call(
        _final_body,
        out_shape=jax.ShapeDtypeStruct((1, 1), jnp.float32),
    )(partials)


def kernel(logits, labels):
    lab3 = labels.astype(jnp.int32).reshape(NB, 1, BLK)
    venc = _dense_call(logits, lab3)
    vflat = venc.reshape(N_ROWS)
    bnd = jnp.linspace(0.0, 1.0, 16, dtype=jnp.float32)
    partials = _make_sc_bin()(vflat, bnd)
    ece = _final_call(partials.reshape(NW, 9, 240))
    return ece.reshape(1)
